# baseline (speedup 1.0000x reference)
import functools

import jax
import jax.numpy as jnp
from jax import lax
from jax.experimental import pallas as pl
from jax.experimental.pallas import tpu as pltpu
from jax.experimental.pallas import tpu_sc as plsc

L = 100000
H = 100000
D = 120
Q = 1024

TILE_L = 2048
N_TILES = pl.cdiv(L, TILE_L)


CHUNK = 128
N_CHUNKS = TILE_L // CHUNK


def _argmax_body(lt_ref, ht_ref, s_ref, idx_ref, hrm_ref, sc_ref, bv_ref, bi_ref):
    i = pl.program_id(0)

    @pl.when(i == 0)
    def _init():
        bv_ref[...] = jnp.full((1, Q), -jnp.inf, dtype=jnp.float32)
        bi_ref[...] = jnp.zeros((1, Q), dtype=jnp.int32)

    hrm_ref[...] = ht_ref[...].T

    def _dot_chunk(k):
        sc_ref[pl.ds(k * CHUNK, CHUNK), :] = lax.dot_general(
            lt_ref[:, pl.ds(k * CHUNK, CHUNK)], s_ref[...],
            dimension_numbers=(((0,), (0,)), ((), ())),
            preferred_element_type=jnp.float32,
        )

    def _commit(tmax, targ):
        upd = tmax > bv_ref[...]
        bv_ref[...] = jnp.where(upd, tmax, bv_ref[...])
        bi_ref[...] = jnp.where(upd, targ, bi_ref[...])

    @pl.when(i < pl.num_programs(0) - 1)
    def _full():
        accs = [jnp.full((8, Q), -jnp.inf, jnp.float32) for _ in range(2)]
        tidxs = [jnp.zeros((8, Q), jnp.int32) for _ in range(2)]
        _dot_chunk(0)
        for k in range(N_CHUNKS):
            if k + 1 < N_CHUNKS:
                _dot_chunk(k + 1)
            for j in range(CHUNK // 8):
                r = k * (CHUNK // 8) + j
                p = r & 1
                x = sc_ref[pl.ds(r * 8, 8), :]
                better = x > accs[p]
                accs[p] = jnp.where(better, x, accs[p])
                tidxs[p] = jnp.where(better, jnp.full((8, Q), r, jnp.int32),
                                     tidxs[p])
        better = (accs[1] > accs[0]) | (
            (accs[1] == accs[0]) & (tidxs[1] < tidxs[0]))
        acc = jnp.where(better, accs[1], accs[0])
        tidx = jnp.where(better, tidxs[1], tidxs[0])
        grow = tidx * 8 + lax.broadcasted_iota(jnp.int32, (8, Q), 0)
        tmax = jnp.max(acc, axis=0, keepdims=True)
        cand = jnp.where(acc == tmax, grow, TILE_L)
        targ = jnp.min(cand, axis=0, keepdims=True) + i * TILE_L
        _commit(tmax, targ)

    @pl.when(i == pl.num_programs(0) - 1)
    def _tail():
        for k in range(N_CHUNKS):
            _dot_chunk(k)
        rows = lax.broadcasted_iota(jnp.int32, (TILE_L, Q), 0)
        scM = jnp.where(rows < (L - i * TILE_L), sc_ref[...], -jnp.inf)
        tmax = jnp.max(scM, axis=0, keepdims=True)
        cand = jnp.where(scM == tmax, rows, TILE_L)
        targ = jnp.min(cand, axis=0, keepdims=True) + i * TILE_L
        _commit(tmax, targ)
        idx_ref[...] = bi_ref[...]


def _argmax_call(lDictT, hDictT, sample):
    return pl.pallas_call(
        _argmax_body,
        grid=(N_TILES,),
        in_specs=[
            pl.BlockSpec((D, TILE_L), lambda i: (0, i)),
            pl.BlockSpec((D, TILE_L), lambda i: (0, i)),
            pl.BlockSpec((D, Q), lambda i: (0, 0)),
        ],
        out_specs=[
            pl.BlockSpec((1, Q), lambda i: (0, 0)),
            pl.BlockSpec((TILE_L, D), lambda i: (i, 0)),
        ],
        out_shape=[
            jax.ShapeDtypeStruct((1, Q), jnp.int32),
            jax.ShapeDtypeStruct((H, D), jnp.float32),
        ],
        scratch_shapes=[
            pltpu.VMEM((TILE_L, Q), jnp.float32),
            pltpu.VMEM((1, Q), jnp.float32),
            pltpu.VMEM((1, Q), jnp.int32),
        ],
    )(lDictT, hDictT, sample)


def _make_gather():
    info = plsc.get_sparse_core_info()
    nw = info.num_cores * info.num_subcores
    b_per_w = Q // nw
    mesh = plsc.VectorSubcoreMesh(core_axis_name="c", subcore_axis_name="s")

    @functools.partial(
        pl.kernel,
        mesh=mesh,
        out_type=jax.ShapeDtypeStruct((Q, D), jnp.float32),
        scratch_types=[
            pltpu.VMEM((b_per_w,), jnp.int32),
            pltpu.VMEM((b_per_w, D), jnp.float32),
            pltpu.SemaphoreType.DMA,
        ],
    )
    def gather(table_hbm, idx_hbm, out_hbm, idx_v, rows_v, sem):
        wid = lax.axis_index("s") * info.num_cores + lax.axis_index("c")
        base = wid * b_per_w
        pltpu.sync_copy(idx_hbm.at[pl.ds(base, b_per_w)], idx_v)
        handles = []
        for c in range(b_per_w // 16):
            vec = idx_v[pl.ds(c * 16, 16)]
            for j in range(16):
                handles.append(pltpu.async_copy(
                    table_hbm.at[vec[j]], rows_v.at[c * 16 + j], sem))
        for h in handles:
            h.wait()
        pltpu.sync_copy(rows_v, out_hbm.at[pl.ds(base, b_per_w)])

    return gather


@functools.lru_cache(maxsize=1)
def _gather_call():
    return _make_gather()


def kernel(sample, lDict, hDict):
    idx, hRM = _argmax_call(lDict.T, hDict.T, sample)
    return _gather_call()(hRM, idx.reshape(Q))


# speedup vs baseline: 1.0123x; 1.0123x over previous
import functools

import jax
import jax.numpy as jnp
from jax import lax
from jax.experimental import pallas as pl
from jax.experimental.pallas import tpu as pltpu
from jax.experimental.pallas import tpu_sc as plsc

L = 100000
H = 100000
D = 120
Q = 1024

TILE_L = 2048
N_TILES = pl.cdiv(L, TILE_L)


CHUNK = 512
N_CHUNKS = TILE_L // CHUNK


def _argmax_body(lt_ref, ht_ref, s_ref, idx_ref, hrm_ref, sc_ref, bv_ref, bi_ref):
    i = pl.program_id(0)

    @pl.when(i == 0)
    def _init():
        bv_ref[...] = jnp.full((1, Q), -jnp.inf, dtype=jnp.float32)
        bi_ref[...] = jnp.zeros((1, Q), dtype=jnp.int32)

    hrm_ref[...] = ht_ref[...].T

    def _dot_chunk(k):
        sc_ref[pl.ds(k * CHUNK, CHUNK), :] = lax.dot_general(
            lt_ref[:, pl.ds(k * CHUNK, CHUNK)], s_ref[...],
            dimension_numbers=(((0,), (0,)), ((), ())),
            preferred_element_type=jnp.float32,
        )

    def _commit(tmax, targ):
        upd = tmax > bv_ref[...]
        bv_ref[...] = jnp.where(upd, tmax, bv_ref[...])
        bi_ref[...] = jnp.where(upd, targ, bi_ref[...])

    @pl.when(i < pl.num_programs(0) - 1)
    def _full():
        accs = [jnp.full((8, Q), -jnp.inf, jnp.float32) for _ in range(2)]
        tidxs = [jnp.zeros((8, Q), jnp.int32) for _ in range(2)]
        _dot_chunk(0)
        for k in range(N_CHUNKS):
            if k + 1 < N_CHUNKS:
                _dot_chunk(k + 1)
            for j in range(CHUNK // 8):
                r = k * (CHUNK // 8) + j
                p = r & 1
                x = sc_ref[pl.ds(r * 8, 8), :]
                better = x > accs[p]
                accs[p] = jnp.where(better, x, accs[p])
                tidxs[p] = jnp.where(better, jnp.full((8, Q), r, jnp.int32),
                                     tidxs[p])
        better = (accs[1] > accs[0]) | (
            (accs[1] == accs[0]) & (tidxs[1] < tidxs[0]))
        acc = jnp.where(better, accs[1], accs[0])
        tidx = jnp.where(better, tidxs[1], tidxs[0])
        grow = tidx * 8 + lax.broadcasted_iota(jnp.int32, (8, Q), 0)
        tmax = jnp.max(acc, axis=0, keepdims=True)
        cand = jnp.where(acc == tmax, grow, TILE_L)
        targ = jnp.min(cand, axis=0, keepdims=True) + i * TILE_L
        _commit(tmax, targ)

    @pl.when(i == pl.num_programs(0) - 1)
    def _tail():
        for k in range(N_CHUNKS):
            _dot_chunk(k)
        rows = lax.broadcasted_iota(jnp.int32, (TILE_L, Q), 0)
        scM = jnp.where(rows < (L - i * TILE_L), sc_ref[...], -jnp.inf)
        tmax = jnp.max(scM, axis=0, keepdims=True)
        cand = jnp.where(scM == tmax, rows, TILE_L)
        targ = jnp.min(cand, axis=0, keepdims=True) + i * TILE_L
        _commit(tmax, targ)
        idx_ref[...] = bi_ref[...]


def _argmax_call(lDictT, hDictT, sample):
    return pl.pallas_call(
        _argmax_body,
        grid=(N_TILES,),
        in_specs=[
            pl.BlockSpec((D, TILE_L), lambda i: (0, i)),
            pl.BlockSpec((D, TILE_L), lambda i: (0, i)),
            pl.BlockSpec((D, Q), lambda i: (0, 0)),
        ],
        out_specs=[
            pl.BlockSpec((1, Q), lambda i: (0, 0)),
            pl.BlockSpec((TILE_L, D), lambda i: (i, 0)),
        ],
        out_shape=[
            jax.ShapeDtypeStruct((1, Q), jnp.int32),
            jax.ShapeDtypeStruct((H, D), jnp.float32),
        ],
        scratch_shapes=[
            pltpu.VMEM((TILE_L, Q), jnp.float32),
            pltpu.VMEM((1, Q), jnp.float32),
            pltpu.VMEM((1, Q), jnp.int32),
        ],
    )(lDictT, hDictT, sample)


def _make_gather():
    info = plsc.get_sparse_core_info()
    nw = info.num_cores * info.num_subcores
    b_per_w = Q // nw
    mesh = plsc.VectorSubcoreMesh(core_axis_name="c", subcore_axis_name="s")

    @functools.partial(
        pl.kernel,
        mesh=mesh,
        out_type=jax.ShapeDtypeStruct((Q, D), jnp.float32),
        scratch_types=[
            pltpu.VMEM((b_per_w,), jnp.int32),
            pltpu.VMEM((b_per_w, D), jnp.float32),
            pltpu.SemaphoreType.DMA,
        ],
    )
    def gather(table_hbm, idx_hbm, out_hbm, idx_v, rows_v, sem):
        wid = lax.axis_index("s") * info.num_cores + lax.axis_index("c")
        base = wid * b_per_w
        pltpu.sync_copy(idx_hbm.at[pl.ds(base, b_per_w)], idx_v)
        handles = []
        for c in range(b_per_w // 16):
            vec = idx_v[pl.ds(c * 16, 16)]
            for j in range(16):
                handles.append(pltpu.async_copy(
                    table_hbm.at[vec[j]], rows_v.at[c * 16 + j], sem))
        for h in handles:
            h.wait()
        pltpu.sync_copy(rows_v, out_hbm.at[pl.ds(base, b_per_w)])

    return gather


@functools.lru_cache(maxsize=1)
def _gather_call():
    return _make_gather()


def kernel(sample, lDict, hDict):
    idx, hRM = _argmax_call(lDict.T, hDict.T, sample)
    return _gather_call()(hRM, idx.reshape(Q))


# speedup vs baseline: 1.1226x; 1.1089x over previous
import functools

import jax
import jax.numpy as jnp
from jax import lax
from jax.experimental import pallas as pl
from jax.experimental.pallas import tpu as pltpu
from jax.experimental.pallas import tpu_sc as plsc

L = 100000
H = 100000
D = 120
Q = 1024

TILE_L = 4096
N_TILES = pl.cdiv(L, TILE_L)


CHUNK = 512
N_CHUNKS = TILE_L // CHUNK


def _argmax_body(lt_ref, ht_ref, s_ref, idx_ref, hrm_ref, sc_ref, bv_ref, bi_ref):
    i = pl.program_id(0)

    @pl.when(i == 0)
    def _init():
        bv_ref[...] = jnp.full((1, Q), -jnp.inf, dtype=jnp.float32)
        bi_ref[...] = jnp.zeros((1, Q), dtype=jnp.int32)

    hrm_ref[...] = ht_ref[...].T

    def _dot_chunk(k):
        sc_ref[pl.ds(k * CHUNK, CHUNK), :] = lax.dot_general(
            lt_ref[:, pl.ds(k * CHUNK, CHUNK)], s_ref[...],
            dimension_numbers=(((0,), (0,)), ((), ())),
            preferred_element_type=jnp.float32,
        )

    def _commit(tmax, targ):
        upd = tmax > bv_ref[...]
        bv_ref[...] = jnp.where(upd, tmax, bv_ref[...])
        bi_ref[...] = jnp.where(upd, targ, bi_ref[...])

    @pl.when(i < pl.num_programs(0) - 1)
    def _full():
        accs = [jnp.full((8, Q), -jnp.inf, jnp.float32) for _ in range(2)]
        tidxs = [jnp.zeros((8, Q), jnp.int32) for _ in range(2)]
        _dot_chunk(0)
        for k in range(N_CHUNKS):
            if k + 1 < N_CHUNKS:
                _dot_chunk(k + 1)
            for j in range(CHUNK // 8):
                r = k * (CHUNK // 8) + j
                p = r & 1
                x = sc_ref[pl.ds(r * 8, 8), :]
                better = x > accs[p]
                accs[p] = jnp.where(better, x, accs[p])
                tidxs[p] = jnp.where(better, jnp.full((8, Q), r, jnp.int32),
                                     tidxs[p])
        better = (accs[1] > accs[0]) | (
            (accs[1] == accs[0]) & (tidxs[1] < tidxs[0]))
        acc = jnp.where(better, accs[1], accs[0])
        tidx = jnp.where(better, tidxs[1], tidxs[0])
        grow = tidx * 8 + lax.broadcasted_iota(jnp.int32, (8, Q), 0)
        tmax = jnp.max(acc, axis=0, keepdims=True)
        cand = jnp.where(acc == tmax, grow, TILE_L)
        targ = jnp.min(cand, axis=0, keepdims=True) + i * TILE_L
        _commit(tmax, targ)

    @pl.when(i == pl.num_programs(0) - 1)
    def _tail():
        for k in range(N_CHUNKS):
            _dot_chunk(k)
        rows = lax.broadcasted_iota(jnp.int32, (TILE_L, Q), 0)
        scM = jnp.where(rows < (L - i * TILE_L), sc_ref[...], -jnp.inf)
        tmax = jnp.max(scM, axis=0, keepdims=True)
        cand = jnp.where(scM == tmax, rows, TILE_L)
        targ = jnp.min(cand, axis=0, keepdims=True) + i * TILE_L
        _commit(tmax, targ)
        idx_ref[...] = bi_ref[...]


def _argmax_call(lDictT, hDictT, sample):
    return pl.pallas_call(
        _argmax_body,
        grid=(N_TILES,),
        in_specs=[
            pl.BlockSpec((D, TILE_L), lambda i: (0, i)),
            pl.BlockSpec((D, TILE_L), lambda i: (0, i)),
            pl.BlockSpec((D, Q), lambda i: (0, 0)),
        ],
        out_specs=[
            pl.BlockSpec((1, Q), lambda i: (0, 0)),
            pl.BlockSpec((TILE_L, D), lambda i: (i, 0)),
        ],
        out_shape=[
            jax.ShapeDtypeStruct((1, Q), jnp.int32),
            jax.ShapeDtypeStruct((H, D), jnp.float32),
        ],
        scratch_shapes=[
            pltpu.VMEM((TILE_L, Q), jnp.float32),
            pltpu.VMEM((1, Q), jnp.float32),
            pltpu.VMEM((1, Q), jnp.int32),
        ],
    )(lDictT, hDictT, sample)


def _make_gather():
    info = plsc.get_sparse_core_info()
    nw = info.num_cores * info.num_subcores
    b_per_w = Q // nw
    mesh = plsc.VectorSubcoreMesh(core_axis_name="c", subcore_axis_name="s")

    @functools.partial(
        pl.kernel,
        mesh=mesh,
        out_type=jax.ShapeDtypeStruct((Q, D), jnp.float32),
        scratch_types=[
            pltpu.VMEM((b_per_w,), jnp.int32),
            pltpu.VMEM((b_per_w, D), jnp.float32),
            pltpu.SemaphoreType.DMA,
        ],
    )
    def gather(table_hbm, idx_hbm, out_hbm, idx_v, rows_v, sem):
        wid = lax.axis_index("s") * info.num_cores + lax.axis_index("c")
        base = wid * b_per_w
        pltpu.sync_copy(idx_hbm.at[pl.ds(base, b_per_w)], idx_v)
        handles = []
        for c in range(b_per_w // 16):
            vec = idx_v[pl.ds(c * 16, 16)]
            for j in range(16):
                handles.append(pltpu.async_copy(
                    table_hbm.at[vec[j]], rows_v.at[c * 16 + j], sem))
        for h in handles:
            h.wait()
        pltpu.sync_copy(rows_v, out_hbm.at[pl.ds(base, b_per_w)])

    return gather


@functools.lru_cache(maxsize=1)
def _gather_call():
    return _make_gather()


def kernel(sample, lDict, hDict):
    idx, hRM = _argmax_call(lDict.T, hDict.T, sample)
    return _gather_call()(hRM, idx.reshape(Q))


# speedup vs baseline: 1.1327x; 1.0090x over previous
import functools

import jax
import jax.numpy as jnp
from jax import lax
from jax.experimental import pallas as pl
from jax.experimental.pallas import tpu as pltpu
from jax.experimental.pallas import tpu_sc as plsc

L = 100000
H = 100000
D = 120
Q = 1024

TILE_L = 5120
N_TILES = pl.cdiv(L, TILE_L)


CHUNK = 512
N_CHUNKS = TILE_L // CHUNK


def _argmax_body(lt_ref, ht_ref, s_ref, idx_ref, hrm_ref, sc_ref, bv_ref, bi_ref):
    i = pl.program_id(0)

    @pl.when(i == 0)
    def _init():
        bv_ref[...] = jnp.full((1, Q), -jnp.inf, dtype=jnp.float32)
        bi_ref[...] = jnp.zeros((1, Q), dtype=jnp.int32)

    hrm_ref[...] = ht_ref[...].T

    def _dot_chunk(k):
        sc_ref[pl.ds(k * CHUNK, CHUNK), :] = lax.dot_general(
            lt_ref[:, pl.ds(k * CHUNK, CHUNK)], s_ref[...],
            dimension_numbers=(((0,), (0,)), ((), ())),
            preferred_element_type=jnp.float32,
        )

    def _commit(tmax, targ):
        upd = tmax > bv_ref[...]
        bv_ref[...] = jnp.where(upd, tmax, bv_ref[...])
        bi_ref[...] = jnp.where(upd, targ, bi_ref[...])

    @pl.when(i < pl.num_programs(0) - 1)
    def _full():
        accs = [jnp.full((8, Q), -jnp.inf, jnp.float32) for _ in range(2)]
        tidxs = [jnp.zeros((8, Q), jnp.int32) for _ in range(2)]
        _dot_chunk(0)
        for k in range(N_CHUNKS):
            if k + 1 < N_CHUNKS:
                _dot_chunk(k + 1)
            for j in range(CHUNK // 8):
                r = k * (CHUNK // 8) + j
                p = r & 1
                x = sc_ref[pl.ds(r * 8, 8), :]
                better = x > accs[p]
                accs[p] = jnp.where(better, x, accs[p])
                tidxs[p] = jnp.where(better, jnp.full((8, Q), r, jnp.int32),
                                     tidxs[p])
        better = (accs[1] > accs[0]) | (
            (accs[1] == accs[0]) & (tidxs[1] < tidxs[0]))
        acc = jnp.where(better, accs[1], accs[0])
        tidx = jnp.where(better, tidxs[1], tidxs[0])
        grow = tidx * 8 + lax.broadcasted_iota(jnp.int32, (8, Q), 0)
        tmax = jnp.max(acc, axis=0, keepdims=True)
        cand = jnp.where(acc == tmax, grow, TILE_L)
        targ = jnp.min(cand, axis=0, keepdims=True) + i * TILE_L
        _commit(tmax, targ)

    @pl.when(i == pl.num_programs(0) - 1)
    def _tail():
        for k in range(N_CHUNKS):
            _dot_chunk(k)
        rows = lax.broadcasted_iota(jnp.int32, (TILE_L, Q), 0)
        scM = jnp.where(rows < (L - i * TILE_L), sc_ref[...], -jnp.inf)
        tmax = jnp.max(scM, axis=0, keepdims=True)
        cand = jnp.where(scM == tmax, rows, TILE_L)
        targ = jnp.min(cand, axis=0, keepdims=True) + i * TILE_L
        _commit(tmax, targ)
        idx_ref[...] = bi_ref[...]


def _argmax_call(lDictT, hDictT, sample):
    return pl.pallas_call(
        _argmax_body,
        grid=(N_TILES,),
        in_specs=[
            pl.BlockSpec((D, TILE_L), lambda i: (0, i)),
            pl.BlockSpec((D, TILE_L), lambda i: (0, i)),
            pl.BlockSpec((D, Q), lambda i: (0, 0)),
        ],
        out_specs=[
            pl.BlockSpec((1, Q), lambda i: (0, 0)),
            pl.BlockSpec((TILE_L, D), lambda i: (i, 0)),
        ],
        out_shape=[
            jax.ShapeDtypeStruct((1, Q), jnp.int32),
            jax.ShapeDtypeStruct((H, D), jnp.float32),
        ],
        scratch_shapes=[
            pltpu.VMEM((TILE_L, Q), jnp.float32),
            pltpu.VMEM((1, Q), jnp.float32),
            pltpu.VMEM((1, Q), jnp.int32),
        ],
    )(lDictT, hDictT, sample)


def _make_gather():
    info = plsc.get_sparse_core_info()
    nw = info.num_cores * info.num_subcores
    b_per_w = Q // nw
    mesh = plsc.VectorSubcoreMesh(core_axis_name="c", subcore_axis_name="s")

    @functools.partial(
        pl.kernel,
        mesh=mesh,
        out_type=jax.ShapeDtypeStruct((Q, D), jnp.float32),
        scratch_types=[
            pltpu.VMEM((b_per_w,), jnp.int32),
            pltpu.VMEM((b_per_w, D), jnp.float32),
            pltpu.SemaphoreType.DMA,
        ],
    )
    def gather(table_hbm, idx_hbm, out_hbm, idx_v, rows_v, sem):
        wid = lax.axis_index("s") * info.num_cores + lax.axis_index("c")
        base = wid * b_per_w
        pltpu.sync_copy(idx_hbm.at[pl.ds(base, b_per_w)], idx_v)
        handles = []
        for c in range(b_per_w // 16):
            vec = idx_v[pl.ds(c * 16, 16)]
            for j in range(16):
                handles.append(pltpu.async_copy(
                    table_hbm.at[vec[j]], rows_v.at[c * 16 + j], sem))
        for h in handles:
            h.wait()
        pltpu.sync_copy(rows_v, out_hbm.at[pl.ds(base, b_per_w)])

    return gather


@functools.lru_cache(maxsize=1)
def _gather_call():
    return _make_gather()


def kernel(sample, lDict, hDict):
    idx, hRM = _argmax_call(lDict.T, hDict.T, sample)
    return _gather_call()(hRM, idx.reshape(Q))
